# Initial kernel scaffold; baseline (speedup 1.0000x reference)
import jax
import jax.numpy as jnp
from jax.experimental import pallas as pl


def kernel(anchor, positive, negative):
    raise NotImplementedError("write your pallas kernel here")


# speedup vs baseline: 1273.5115x; 1273.5115x over previous
import functools

import jax
import jax.numpy as jnp
from jax import lax
from jax.experimental import pallas as pl
from jax.experimental.pallas import tpu as pltpu
from jax.experimental.pallas import tpu_sc as plsc

H = 2047
NP = 1023
NB = 341
GPB = 341
NW = 32

_MESH = plsc.VectorSubcoreMesh(core_axis_name="c", subcore_axis_name="s")
_PARAMS = pltpu.CompilerParams(
    use_tc_tiling_on_sc=False, needs_layout_passes=False)


def _worker_id():
    return lax.axis_index("s") * 2 + lax.axis_index("c")


@functools.partial(
    pl.kernel,
    mesh=_MESH,
    out_type=[
        jax.ShapeDtypeStruct((9, NB, GPB), jnp.float32),
        jax.ShapeDtypeStruct((9, NB, GPB), jnp.float32),
    ],
    scratch_types=[
        pltpu.VMEM((7, H), jnp.float32),
        pltpu.VMEM((7, H), jnp.float32),
        pltpu.VMEM((7, H), jnp.float32),
        pltpu.VMEM((GPB,), jnp.float32),
        pltpu.VMEM((GPB,), jnp.float32),
        pltpu.SemaphoreType.DMA,
    ],
    compiler_params=_PARAMS,
)
def _phase1(a_hbm, p_hbm, n_hbm, red_p, red_n,
            a_buf, p_buf, n_buf, out_p, out_n, sem):
    wid = _worker_id()
    lo = wid * NB // NW
    hi = (wid + 1) * NB // NW
    lanes = lax.iota(jnp.int32, 16)

    def unit_body(b, carry):
        row0 = 6 * b
        cpa = pltpu.make_async_copy(a_hbm.at[pl.ds(row0, 7)], a_buf, sem)
        cpp = pltpu.make_async_copy(p_hbm.at[pl.ds(row0, 7)], p_buf, sem)
        cpn = pltpu.make_async_copy(n_hbm.at[pl.ds(row0, 7)], n_buf, sem)
        cpa.start()
        cpp.start()
        cpn.start()
        cpa.wait()
        cpp.wait()
        cpn.wait()

        def e_body(e, ecarry):
            ki = e // 3
            kj = e - 3 * ki

            def v_body(v, vcarry):
                start = jnp.minimum(v * 16, GPB - 16)
                base = 9 * (start + lanes)

                def gather(j):
                    p = base + j
                    row = lax.div(p, 1023)
                    col = p - row * 1023
                    brow = 2 * row + ki
                    bcol = kj + 2 * col
                    av = plsc.load_gather(a_buf, [brow, bcol])
                    pv = plsc.load_gather(p_buf, [brow, bcol])
                    nv = plsc.load_gather(n_buf, [brow, bcol])
                    return jnp.abs(av - pv), pv, jnp.abs(av - nv), nv

                dp, cp, dn, cn = gather(0)
                bdp, bcp, wdp, wcp = dp, cp, dp, cp
                bdn, bcn, wdn, wcn = dn, cn, dn, cn
                for j in range(1, 9):
                    dp, cp, dn, cn = gather(j)
                    m = dp < bdp
                    bdp = jnp.where(m, dp, bdp)
                    bcp = jnp.where(m, cp, bcp)
                    m = dp > wdp
                    wdp = jnp.where(m, dp, wdp)
                    wcp = jnp.where(m, cp, wcp)
                    m = dn < bdn
                    bdn = jnp.where(m, dn, bdn)
                    bcn = jnp.where(m, cn, bcn)
                    m = dn > wdn
                    wdn = jnp.where(m, dn, wdn)
                    wcn = jnp.where(m, cn, wcn)
                out_p[pl.ds(start, 16)] = bcp + wcp
                out_n[pl.ds(start, 16)] = bcn + wcn
                return vcarry

            lax.fori_loop(0, 22, v_body, 0)
            pltpu.sync_copy(out_p, red_p.at[e, b])
            pltpu.sync_copy(out_n, red_n.at[e, b])
            return ecarry

        lax.fori_loop(0, 9, e_body, 0)
        return carry

    lax.fori_loop(lo, hi, unit_body, 0)


@functools.partial(
    pl.kernel,
    mesh=_MESH,
    out_type=[
        jax.ShapeDtypeStruct((H, H), jnp.float32),
        jax.ShapeDtypeStruct((H, H), jnp.float32),
    ],
    scratch_types=[
        pltpu.VMEM((NP,), jnp.float32),
        pltpu.VMEM((NP,), jnp.float32),
        pltpu.VMEM((H,), jnp.float32),
        pltpu.VMEM((H,), jnp.float32),
        pltpu.SemaphoreType.DMA,
    ],
    compiler_params=_PARAMS,
)
def _phase2(sp_hbm, sn_hbm, op_hbm, on_hbm, s_p, s_n, d_p, d_n, sem):
    wid = _worker_id()
    lo = wid * NP // NW
    hi = (wid + 1) * NP // NW
    lanes = lax.iota(jnp.int32, 16)

    def row_body(r, carry):
        cpa = pltpu.make_async_copy(sp_hbm.at[r], s_p, sem)
        cpb = pltpu.make_async_copy(sn_hbm.at[r], s_n, sem)
        cpa.start()
        cpb.start()
        cpa.wait()
        cpb.wait()

        def col_body(m, ccarry):
            start = jnp.minimum(m * 16, H - 16)
            idx = jnp.minimum((start + lanes) >> 1, NP - 1)
            d_p[pl.ds(start, 16)] = plsc.load_gather(s_p, [idx])
            d_n[pl.ds(start, 16)] = plsc.load_gather(s_n, [idx])
            return ccarry

        lax.fori_loop(0, 128, col_body, 0)
        pltpu.sync_copy(d_p, op_hbm.at[2 * r])
        pltpu.sync_copy(d_p, op_hbm.at[2 * r + 1])
        pltpu.sync_copy(d_n, on_hbm.at[2 * r])
        pltpu.sync_copy(d_n, on_hbm.at[2 * r + 1])

        @pl.when(r == NP - 1)
        def _():
            pltpu.sync_copy(d_p, op_hbm.at[H - 1])
            pltpu.sync_copy(d_n, on_hbm.at[H - 1])

        return carry

    lax.fori_loop(lo, hi, row_body, 0)


def kernel(anchor, positive, negative):
    red_p, red_n = _phase1(anchor, positive, negative)
    out_p, out_n = _phase2(red_p.reshape(NP, NP), red_n.reshape(NP, NP))
    return (out_p, out_n)


# speedup vs baseline: 1394.8522x; 1.0953x over previous
import functools

import jax
import jax.numpy as jnp
from jax import lax
from jax.experimental import pallas as pl
from jax.experimental.pallas import tpu as pltpu
from jax.experimental.pallas import tpu_sc as plsc

H = 2047
NP = 1023
NB = 341
GPB = 341
NW = 32

_MESH = plsc.VectorSubcoreMesh(core_axis_name="c", subcore_axis_name="s")
_PARAMS = pltpu.CompilerParams(
    use_tc_tiling_on_sc=False, needs_layout_passes=False)


def _worker_id():
    return lax.axis_index("s") * 2 + lax.axis_index("c")


@functools.partial(
    pl.kernel,
    mesh=_MESH,
    out_type=[
        jax.ShapeDtypeStruct((9, NB, GPB), jnp.float32),
        jax.ShapeDtypeStruct((9, NB, GPB), jnp.float32),
    ],
    scratch_types=[
        pltpu.VMEM((7, H), jnp.float32),
        pltpu.VMEM((7, H), jnp.float32),
        pltpu.VMEM((7, H), jnp.float32),
        pltpu.VMEM((GPB,), jnp.float32),
        pltpu.VMEM((GPB,), jnp.float32),
        pltpu.SemaphoreType.DMA,
    ],
    compiler_params=_PARAMS,
)
def _phase1(a_hbm, p_hbm, n_hbm, red_p, red_n,
            a_buf, p_buf, n_buf, out_p, out_n, sem):
    wid = _worker_id()
    lo = wid * NB // NW
    hi = (wid + 1) * NB // NW
    lanes = lax.iota(jnp.int32, 16)

    def unit_body(b, carry):
        row0 = 6 * b
        cpa = pltpu.make_async_copy(a_hbm.at[pl.ds(row0, 7)], a_buf, sem)
        cpp = pltpu.make_async_copy(p_hbm.at[pl.ds(row0, 7)], p_buf, sem)
        cpn = pltpu.make_async_copy(n_hbm.at[pl.ds(row0, 7)], n_buf, sem)
        cpa.start()
        cpp.start()
        cpn.start()
        cpa.wait()
        cpp.wait()
        cpn.wait()

        def e_body(e, ecarry):
            ki = e // 3
            kj = e - 3 * ki

            def v_body(v, vcarry):
                start = jnp.minimum(v * 16, GPB - 16)
                base = 9 * (start + lanes)

                def gather(j):
                    p = base + j
                    row = lax.div(p, 1023)
                    col = p - row * 1023
                    brow = 2 * row + ki
                    bcol = kj + 2 * col
                    av = plsc.load_gather(a_buf, [brow, bcol])
                    pv = plsc.load_gather(p_buf, [brow, bcol])
                    nv = plsc.load_gather(n_buf, [brow, bcol])
                    return jnp.abs(av - pv), pv, jnp.abs(av - nv), nv

                dp, cp, dn, cn = gather(0)
                bdp, bcp, wdp, wcp = dp, cp, dp, cp
                bdn, bcn, wdn, wcn = dn, cn, dn, cn
                for j in range(1, 9):
                    dp, cp, dn, cn = gather(j)
                    m = dp < bdp
                    bdp = jnp.where(m, dp, bdp)
                    bcp = jnp.where(m, cp, bcp)
                    m = dp > wdp
                    wdp = jnp.where(m, dp, wdp)
                    wcp = jnp.where(m, cp, wcp)
                    m = dn < bdn
                    bdn = jnp.where(m, dn, bdn)
                    bcn = jnp.where(m, cn, bcn)
                    m = dn > wdn
                    wdn = jnp.where(m, dn, wdn)
                    wcn = jnp.where(m, cn, wcn)
                out_p[pl.ds(start, 16)] = bcp + wcp
                out_n[pl.ds(start, 16)] = bcn + wcn
                return vcarry

            lax.fori_loop(0, 22, v_body, 0)
            pltpu.sync_copy(out_p, red_p.at[e, b])
            pltpu.sync_copy(out_n, red_n.at[e, b])
            return ecarry

        lax.fori_loop(0, 9, e_body, 0)
        return carry

    lax.fori_loop(lo, hi, unit_body, 0)


@functools.partial(
    pl.kernel,
    mesh=_MESH,
    out_type=[
        jax.ShapeDtypeStruct((H, H), jnp.float32),
        jax.ShapeDtypeStruct((H, H), jnp.float32),
    ],
    scratch_types=[
        pltpu.VMEM((8, NP), jnp.float32),
        pltpu.VMEM((8, NP), jnp.float32),
        pltpu.VMEM((16, H), jnp.float32),
        pltpu.VMEM((16, H), jnp.float32),
        pltpu.SemaphoreType.DMA,
    ],
    compiler_params=_PARAMS,
)
def _phase2(sp_hbm, sn_hbm, op_hbm, on_hbm, s_p, s_n, d_p, d_n, sem):
    wid = _worker_id()
    lanes = lax.iota(jnp.int32, 16)

    def blk_body(k, carry):
        r0 = jnp.minimum(8 * k, NP - 8)
        cpa = pltpu.make_async_copy(sp_hbm.at[pl.ds(r0, 8)], s_p, sem)
        cpb = pltpu.make_async_copy(sn_hbm.at[pl.ds(r0, 8)], s_n, sem)
        cpa.start()
        cpb.start()
        cpa.wait()
        cpb.wait()

        def col_body(m, ccarry):
            start = jnp.minimum(m * 16, H - 16)
            idx = jnp.minimum((start + lanes) >> 1, NP - 1)
            for q in range(8):
                qv = jnp.full((16,), q, jnp.int32)
                vp = plsc.load_gather(s_p, [qv, idx])
                vn = plsc.load_gather(s_n, [qv, idx])
                d_p[2 * q, pl.ds(start, 16)] = vp
                d_p[2 * q + 1, pl.ds(start, 16)] = vp
                d_n[2 * q, pl.ds(start, 16)] = vn
                d_n[2 * q + 1, pl.ds(start, 16)] = vn
            return ccarry

        lax.fori_loop(0, 128, col_body, 0)
        pltpu.sync_copy(d_p, op_hbm.at[pl.ds(2 * r0, 16)])
        pltpu.sync_copy(d_n, on_hbm.at[pl.ds(2 * r0, 16)])

        @pl.when(k == 127)
        def _():
            pltpu.sync_copy(d_p.at[15], op_hbm.at[H - 1])
            pltpu.sync_copy(d_n.at[15], on_hbm.at[H - 1])

        return carry

    lax.fori_loop(wid * 4, wid * 4 + 4, blk_body, 0)


def kernel(anchor, positive, negative):
    red_p, red_n = _phase1(anchor, positive, negative)
    out_p, out_n = _phase2(red_p.reshape(NP, NP), red_n.reshape(NP, NP))
    return (out_p, out_n)


# speedup vs baseline: 1630.8859x; 1.1692x over previous
import functools

import jax
import jax.numpy as jnp
from jax import lax
from jax.experimental import pallas as pl
from jax.experimental.pallas import tpu as pltpu
from jax.experimental.pallas import tpu_sc as plsc

H = 2047
NP = 1023
NB = 341
GPB = 341
NW = 32

_MESH = plsc.VectorSubcoreMesh(core_axis_name="c", subcore_axis_name="s")
_PARAMS = pltpu.CompilerParams(
    use_tc_tiling_on_sc=False, needs_layout_passes=False)


def _worker_id():
    return lax.axis_index("s") * 2 + lax.axis_index("c")


@functools.partial(
    pl.kernel,
    mesh=_MESH,
    out_type=[
        jax.ShapeDtypeStruct((9, NB, GPB), jnp.float32),
        jax.ShapeDtypeStruct((9, NB, GPB), jnp.float32),
    ],
    scratch_types=[
        pltpu.VMEM((7, H), jnp.float32),
        pltpu.VMEM((7, H), jnp.float32),
        pltpu.VMEM((7, H), jnp.float32),
        pltpu.VMEM((7, H), jnp.float32),
        pltpu.VMEM((7, H), jnp.float32),
        pltpu.VMEM((7, H), jnp.float32),
        pltpu.VMEM((9, GPB), jnp.float32),
        pltpu.VMEM((9, GPB), jnp.float32),
        pltpu.SemaphoreType.DMA,
        pltpu.SemaphoreType.DMA,
        pltpu.SemaphoreType.DMA,
    ],
    compiler_params=_PARAMS,
)
def _phase1(a_hbm, p_hbm, n_hbm, red_p, red_n,
            a0, p0, n0, a1, p1, n1, out_p, out_n, sem0, sem1, semo):
    wid = _worker_id()
    lo = wid * NB // NW
    hi = (wid + 1) * NB // NW
    nu = hi - lo
    lanes = lax.iota(jnp.int32, 16)
    hbms = (a_hbm, p_hbm, n_hbm)
    bufs0 = (a0, p0, n0)
    bufs1 = (a1, p1, n1)

    def _start(b, bufs, sem):
        row0 = 6 * b
        for src, dst in zip(hbms, bufs):
            pltpu.make_async_copy(src.at[pl.ds(row0, 7)], dst, sem).start()

    def _drain(bufs, sem):
        for src, dst in zip(hbms, bufs):
            pltpu.make_async_copy(src.at[pl.ds(0, 7)], dst, sem).wait()

    def _compute(b, a_buf, p_buf, n_buf):
        def v_body(v, vcarry):
            start = jnp.minimum(v * 16, GPB - 16)
            base = 9 * (start + lanes)
            brows = []
            bcols = []
            for j in range(9):
                p = base + j
                row = (p >= 1023).astype(jnp.int32) + (p >= 2046).astype(
                    jnp.int32)
                brows.append(row + row)
                bcols.append((p - row * 1023) * 2)
            for e in range(9):
                ki = e // 3
                kj = e - 3 * ki

                def gather(j):
                    br = brows[j] + ki
                    bc = bcols[j] + kj
                    av = plsc.load_gather(a_buf, [br, bc])
                    pv = plsc.load_gather(p_buf, [br, bc])
                    nv = plsc.load_gather(n_buf, [br, bc])
                    return jnp.abs(av - pv), pv, jnp.abs(av - nv), nv

                dp, cp, dn, cn = gather(0)
                bdp, bcp, wdp, wcp = dp, cp, dp, cp
                bdn, bcn, wdn, wcn = dn, cn, dn, cn
                for j in range(1, 9):
                    dp, cp, dn, cn = gather(j)
                    m = dp < bdp
                    bdp = jnp.where(m, dp, bdp)
                    bcp = jnp.where(m, cp, bcp)
                    m = dp > wdp
                    wdp = jnp.where(m, dp, wdp)
                    wcp = jnp.where(m, cp, wcp)
                    m = dn < bdn
                    bdn = jnp.where(m, dn, bdn)
                    bcn = jnp.where(m, cn, bcn)
                    m = dn > wdn
                    wdn = jnp.where(m, dn, wdn)
                    wcn = jnp.where(m, cn, wcn)
                out_p[e, pl.ds(start, 16)] = bcp + wcp
                out_n[e, pl.ds(start, 16)] = bcn + wcn
            return vcarry

        lax.fori_loop(0, 22, v_body, 0)
        outs = []
        for e in range(9):
            outs.append(
                pltpu.make_async_copy(out_p.at[e], red_p.at[e, b], semo))
            outs.append(
                pltpu.make_async_copy(out_n.at[e], red_n.at[e, b], semo))
        for c in outs:
            c.start()
        for c in outs:
            c.wait()

    @pl.when(nu > 0)
    def _():
        _start(lo, bufs0, sem0)

    def pair_body(i, carry):
        b0 = lo + 2 * i
        _drain(bufs0, sem0)

        @pl.when(b0 + 1 < hi)
        def _():
            _start(b0 + 1, bufs1, sem1)

        _compute(b0, a0, p0, n0)

        @pl.when(b0 + 1 < hi)
        def _():
            _drain(bufs1, sem1)

            @pl.when(b0 + 2 < hi)
            def _():
                _start(b0 + 2, bufs0, sem0)

            _compute(b0 + 1, a1, p1, n1)

        return carry

    lax.fori_loop(0, (nu + 1) // 2, pair_body, 0)


@functools.partial(
    pl.kernel,
    mesh=_MESH,
    out_type=[
        jax.ShapeDtypeStruct((H, H), jnp.float32),
        jax.ShapeDtypeStruct((H, H), jnp.float32),
    ],
    scratch_types=[
        pltpu.VMEM((8, NP), jnp.float32),
        pltpu.VMEM((8, NP), jnp.float32),
        pltpu.VMEM((16, H), jnp.float32),
        pltpu.VMEM((16, H), jnp.float32),
        pltpu.SemaphoreType.DMA,
    ],
    compiler_params=_PARAMS,
)
def _phase2(sp_hbm, sn_hbm, op_hbm, on_hbm, s_p, s_n, d_p, d_n, sem):
    wid = _worker_id()
    lanes = lax.iota(jnp.int32, 16)

    def blk_body(k, carry):
        r0 = jnp.minimum(8 * k, NP - 8)
        cpa = pltpu.make_async_copy(sp_hbm.at[pl.ds(r0, 8)], s_p, sem)
        cpb = pltpu.make_async_copy(sn_hbm.at[pl.ds(r0, 8)], s_n, sem)
        cpa.start()
        cpb.start()
        cpa.wait()
        cpb.wait()

        def col_body(m, ccarry):
            start = jnp.minimum(m * 16, H - 16)
            idx = jnp.minimum((start + lanes) >> 1, NP - 1)
            for q in range(8):
                qv = jnp.full((16,), q, jnp.int32)
                vp = plsc.load_gather(s_p, [qv, idx])
                vn = plsc.load_gather(s_n, [qv, idx])
                d_p[2 * q, pl.ds(start, 16)] = vp
                d_p[2 * q + 1, pl.ds(start, 16)] = vp
                d_n[2 * q, pl.ds(start, 16)] = vn
                d_n[2 * q + 1, pl.ds(start, 16)] = vn
            return ccarry

        lax.fori_loop(0, 128, col_body, 0)
        pltpu.sync_copy(d_p, op_hbm.at[pl.ds(2 * r0, 16)])
        pltpu.sync_copy(d_n, on_hbm.at[pl.ds(2 * r0, 16)])

        @pl.when(k == 127)
        def _():
            pltpu.sync_copy(d_p.at[15], op_hbm.at[H - 1])
            pltpu.sync_copy(d_n.at[15], on_hbm.at[H - 1])

        return carry

    lax.fori_loop(wid * 4, wid * 4 + 4, blk_body, 0)


def kernel(anchor, positive, negative):
    red_p, red_n = _phase1(anchor, positive, negative)
    out_p, out_n = _phase2(red_p.reshape(NP, NP), red_n.reshape(NP, NP))
    return (out_p, out_n)


# speedup vs baseline: 1696.1282x; 1.0400x over previous
import functools

import jax
import jax.numpy as jnp
from jax import lax
from jax.experimental import pallas as pl
from jax.experimental.pallas import tpu as pltpu
from jax.experimental.pallas import tpu_sc as plsc

H = 2047
NP = 1023
NB = 341
GPB = 341
NW = 32

_MESH = plsc.VectorSubcoreMesh(core_axis_name="c", subcore_axis_name="s")
_PARAMS = pltpu.CompilerParams(
    use_tc_tiling_on_sc=False, needs_layout_passes=False)


def _worker_id():
    return lax.axis_index("s") * 2 + lax.axis_index("c")


@functools.partial(
    pl.kernel,
    mesh=_MESH,
    out_type=[
        jax.ShapeDtypeStruct((9, NB, GPB), jnp.float32),
        jax.ShapeDtypeStruct((9, NB, GPB), jnp.float32),
    ],
    scratch_types=[
        pltpu.VMEM((7, H), jnp.float32),
        pltpu.VMEM((7, H), jnp.float32),
        pltpu.VMEM((7, H), jnp.float32),
        pltpu.VMEM((7, H), jnp.float32),
        pltpu.VMEM((7, H), jnp.float32),
        pltpu.VMEM((7, H), jnp.float32),
        pltpu.VMEM((9, GPB), jnp.float32),
        pltpu.VMEM((9, GPB), jnp.float32),
        pltpu.VMEM((9, GPB), jnp.float32),
        pltpu.VMEM((9, GPB), jnp.float32),
        pltpu.SemaphoreType.DMA,
        pltpu.SemaphoreType.DMA,
        pltpu.SemaphoreType.DMA,
        pltpu.SemaphoreType.DMA,
    ],
    compiler_params=_PARAMS,
)
def _phase1(a_hbm, p_hbm, n_hbm, red_p, red_n,
            a0, p0, n0, a1, p1, n1,
            out_p0, out_n0, out_p1, out_n1, sem0, sem1, semo0, semo1):
    wid = _worker_id()
    lo = wid * NB // NW
    hi = (wid + 1) * NB // NW
    nu = hi - lo
    lanes = lax.iota(jnp.int32, 16)
    hbms = (a_hbm, p_hbm, n_hbm)
    bufs0 = (a0, p0, n0)
    bufs1 = (a1, p1, n1)

    def _start(b, bufs, sem):
        row0 = 6 * b
        for src, dst in zip(hbms, bufs):
            pltpu.make_async_copy(src.at[pl.ds(row0, 7)], dst, sem).start()

    def _drain(bufs, sem):
        for src, dst in zip(hbms, bufs):
            pltpu.make_async_copy(src.at[pl.ds(0, 7)], dst, sem).wait()

    def _drain_outs(b, out_p, out_n, semo):
        for e in range(9):
            pltpu.make_async_copy(out_p.at[e], red_p.at[e, b], semo).wait()
            pltpu.make_async_copy(out_n.at[e], red_n.at[e, b], semo).wait()

    def _compute(b, a_buf, p_buf, n_buf, out_p, out_n, semo):
        def v_body(v, vcarry):
            start = jnp.minimum(v * 16, GPB - 16)
            base = 9 * (start + lanes)
            brows = []
            bcols = []
            for j in range(9):
                p = base + j
                row = (p >= 1023).astype(jnp.int32) + (p >= 2046).astype(
                    jnp.int32)
                brows.append(row + row)
                bcols.append((p - row * 1023) * 2)
            for e in range(9):
                ki = e // 3
                kj = e - 3 * ki

                def gather(j):
                    br = brows[j] + ki
                    bc = bcols[j] + kj
                    av = plsc.load_gather(a_buf, [br, bc])
                    pv = plsc.load_gather(p_buf, [br, bc])
                    nv = plsc.load_gather(n_buf, [br, bc])
                    return jnp.abs(av - pv), pv, jnp.abs(av - nv), nv

                dp, cp, dn, cn = gather(0)
                bdp, bcp, wdp, wcp = dp, cp, dp, cp
                bdn, bcn, wdn, wcn = dn, cn, dn, cn
                for j in range(1, 9):
                    dp, cp, dn, cn = gather(j)
                    m = dp < bdp
                    bdp = jnp.where(m, dp, bdp)
                    bcp = jnp.where(m, cp, bcp)
                    m = dp > wdp
                    wdp = jnp.where(m, dp, wdp)
                    wcp = jnp.where(m, cp, wcp)
                    m = dn < bdn
                    bdn = jnp.where(m, dn, bdn)
                    bcn = jnp.where(m, cn, bcn)
                    m = dn > wdn
                    wdn = jnp.where(m, dn, wdn)
                    wcn = jnp.where(m, cn, wcn)
                out_p[e, pl.ds(start, 16)] = bcp + wcp
                out_n[e, pl.ds(start, 16)] = bcn + wcn
            return vcarry

        lax.fori_loop(0, 22, v_body, 0)
        for e in range(9):
            pltpu.make_async_copy(out_p.at[e], red_p.at[e, b], semo).start()
            pltpu.make_async_copy(out_n.at[e], red_n.at[e, b], semo).start()

    @pl.when(nu > 0)
    def _():
        _start(lo, bufs0, sem0)

    def pair_body(i, carry):
        b0 = lo + 2 * i
        _drain(bufs0, sem0)

        @pl.when(b0 + 1 < hi)
        def _():
            _start(b0 + 1, bufs1, sem1)

        @pl.when(i > 0)
        def _():
            _drain_outs(b0, out_p0, out_n0, semo0)

        _compute(b0, a0, p0, n0, out_p0, out_n0, semo0)

        @pl.when(b0 + 1 < hi)
        def _():
            _drain(bufs1, sem1)

            @pl.when(b0 + 2 < hi)
            def _():
                _start(b0 + 2, bufs0, sem0)

            @pl.when(i > 0)
            def _():
                _drain_outs(b0, out_p1, out_n1, semo1)

            _compute(b0 + 1, a1, p1, n1, out_p1, out_n1, semo1)

        return carry

    lax.fori_loop(0, (nu + 1) // 2, pair_body, 0)

    @pl.when(nu >= 1)
    def _():
        _drain_outs(lo, out_p0, out_n0, semo0)

    @pl.when(nu >= 2)
    def _():
        _drain_outs(lo, out_p1, out_n1, semo1)


@functools.partial(
    pl.kernel,
    mesh=_MESH,
    out_type=[
        jax.ShapeDtypeStruct((H, H), jnp.float32),
        jax.ShapeDtypeStruct((H, H), jnp.float32),
    ],
    scratch_types=[
        pltpu.VMEM((4, NP), jnp.float32),
        pltpu.VMEM((4, NP), jnp.float32),
        pltpu.VMEM((4, NP), jnp.float32),
        pltpu.VMEM((4, NP), jnp.float32),
        pltpu.VMEM((8, H), jnp.float32),
        pltpu.VMEM((8, H), jnp.float32),
        pltpu.VMEM((8, H), jnp.float32),
        pltpu.VMEM((8, H), jnp.float32),
        pltpu.SemaphoreType.DMA,
        pltpu.SemaphoreType.DMA,
        pltpu.SemaphoreType.DMA,
        pltpu.SemaphoreType.DMA,
    ],
    compiler_params=_PARAMS,
)
def _phase2(sp_hbm, sn_hbm, op_hbm, on_hbm,
            sp0, sn0, sp1, sn1, dp0, dn0, dp1, dn1,
            semi0, semi1, semo0, semo1):
    wid = _worker_id()
    lanes = lax.iota(jnp.int32, 16)
    lo = wid * 8

    def _r0(k):
        return jnp.minimum(4 * k, NP - 4)

    def _start_in(k, s_p, s_n, semi):
        r0 = _r0(k)
        pltpu.make_async_copy(sp_hbm.at[pl.ds(r0, 4)], s_p, semi).start()
        pltpu.make_async_copy(sn_hbm.at[pl.ds(r0, 4)], s_n, semi).start()

    def _drain_in(s_p, s_n, semi):
        pltpu.make_async_copy(sp_hbm.at[pl.ds(0, 4)], s_p, semi).wait()
        pltpu.make_async_copy(sn_hbm.at[pl.ds(0, 4)], s_n, semi).wait()

    def _drain_out(d_p, d_n, semo):
        pltpu.make_async_copy(d_p, op_hbm.at[pl.ds(0, 8)], semo).wait()
        pltpu.make_async_copy(d_n, on_hbm.at[pl.ds(0, 8)], semo).wait()

    def _compute(k, s_p, s_n, d_p, d_n, semo):
        def col_body(m, ccarry):
            start = jnp.minimum(m * 16, H - 16)
            idx = jnp.minimum((start + lanes) >> 1, NP - 1)
            for q in range(4):
                qv = jnp.full((16,), q, jnp.int32)
                vp = plsc.load_gather(s_p, [qv, idx])
                vn = plsc.load_gather(s_n, [qv, idx])
                d_p[2 * q, pl.ds(start, 16)] = vp
                d_p[2 * q + 1, pl.ds(start, 16)] = vp
                d_n[2 * q, pl.ds(start, 16)] = vn
                d_n[2 * q + 1, pl.ds(start, 16)] = vn
            return ccarry

        lax.fori_loop(0, 128, col_body, 0)
        r0 = _r0(k)
        pltpu.make_async_copy(d_p, op_hbm.at[pl.ds(2 * r0, 8)], semo).start()
        pltpu.make_async_copy(d_n, on_hbm.at[pl.ds(2 * r0, 8)], semo).start()

        @pl.when(k == 255)
        def _():
            pltpu.make_async_copy(d_p.at[7], op_hbm.at[H - 1], semo).start()
            pltpu.make_async_copy(d_n.at[7], on_hbm.at[H - 1], semo).start()

    _start_in(lo, sp0, sn0, semi0)

    def pair_body(i, carry):
        k0 = lo + 2 * i
        _drain_in(sp0, sn0, semi0)
        _start_in(k0 + 1, sp1, sn1, semi1)

        @pl.when(i > 0)
        def _():
            _drain_out(dp0, dn0, semo0)

        _compute(k0, sp0, sn0, dp0, dn0, semo0)
        _drain_in(sp1, sn1, semi1)

        @pl.when(k0 + 2 < lo + 8)
        def _():
            _start_in(k0 + 2, sp0, sn0, semi0)

        @pl.when(i > 0)
        def _():
            _drain_out(dp1, dn1, semo1)

        _compute(k0 + 1, sp1, sn1, dp1, dn1, semo1)
        return carry

    lax.fori_loop(0, 4, pair_body, 0)
    _drain_out(dp0, dn0, semo0)
    _drain_out(dp1, dn1, semo1)

    @pl.when(wid == NW - 1)
    def _():
        pltpu.make_async_copy(dp1.at[7], op_hbm.at[H - 1], semo1).wait()
        pltpu.make_async_copy(dn1.at[7], on_hbm.at[H - 1], semo1).wait()


def kernel(anchor, positive, negative):
    red_p, red_n = _phase1(anchor, positive, negative)
    out_p, out_n = _phase2(red_p.reshape(NP, NP), red_n.reshape(NP, NP))
    return (out_p, out_n)
